# baseline (speedup 1.0000x reference)
import functools

import jax
import jax.numpy as jnp
from jax import lax
from jax.experimental import pallas as pl
from jax.experimental.pallas import tpu as pltpu
from jax.experimental.pallas import tpu_sc as plsc

_B = 4096
_S = 200
_H = 128
_T = _B * _S
_BB = 128

_NC = 2
_NS = 16
_NW = _NC * _NS
_PW = _T // _NW
_C = 128
_NCHUNK = _PW // _C


def _prep_body(idx_ref, emb_int_ref, emb_pos_ref, flat_ref, comb_ref):
    i = pl.program_id(0)
    pos = emb_pos_ref[...]

    @pl.when(i == 0)
    def _():
        comb_ref[...] = pos[None, :, :] + emb_int_ref[...][:, None, :]

    flat_ref[...] = idx_ref[...] * _S + lax.broadcasted_iota(jnp.int32, (_BB, _S), 1)


def _prep(interaction, emb_interaction, emb_position):
    return pl.pallas_call(
        _prep_body,
        grid=(_B // _BB,),
        in_specs=[
            pl.BlockSpec((_BB, _S), lambda i: (i, 0)),
            pl.BlockSpec((3, _H), lambda i: (0, 0)),
            pl.BlockSpec((_S, _H), lambda i: (0, 0)),
        ],
        out_specs=[
            pl.BlockSpec((_BB, _S), lambda i: (i, 0)),
            pl.BlockSpec((3, _S, _H), lambda i: (0, 0, 0)),
        ],
        out_shape=[
            jax.ShapeDtypeStruct((_B, _S), jnp.int32),
            jax.ShapeDtypeStruct((3, _S, _H), jnp.float32),
        ],
    )(interaction, emb_interaction, emb_position)


_NBUF = 5
_LOOK = 1


def _out_slice(out_hbm, base, g):
    return out_hbm.at[pl.ds(base + g * _C, _C)]


def _sc_body(comb_hbm, flat_hbm, out_hbm, idx_all, comb_sp, *scratch):
    rows = scratch[:_NBUF]
    gsem = scratch[_NBUF : 2 * _NBUF]
    osem = scratch[2 * _NBUF : 3 * _NBUF]
    sid = lax.axis_index("s")
    wid = sid * _NC + lax.axis_index("c")
    base = wid * _PW

    @pl.when(sid == 0)
    def _():
        pltpu.sync_copy(comb_hbm, comb_sp)

    pltpu.sync_copy(flat_hbm.at[pl.ds(wid * _NCHUNK, _NCHUNK)], idx_all)
    plsc.subcore_barrier()

    for b in range(_LOOK):
        pltpu.async_copy(comb_sp.at[idx_all.at[b]], rows[b], gsem[b])

    def outer(i, carry):
        for b in range(_NBUF):
            g = i * _NBUF + b
            pltpu.make_async_copy(comb_sp.at[idx_all.at[g]], rows[b], gsem[b]).wait()
            pltpu.async_copy(rows[b], _out_slice(out_hbm, base, g), osem[b])
            b2 = (b + _LOOK) % _NBUF

            @pl.when(g + _LOOK < _NCHUNK)
            def _():
                @pl.when(g + _LOOK >= _NBUF)
                def _():
                    pltpu.make_async_copy(
                        rows[b2], _out_slice(out_hbm, base, g + _LOOK - _NBUF), osem[b2]
                    ).wait()

                pltpu.async_copy(
                    comb_sp.at[idx_all.at[g + _LOOK]], rows[b2], gsem[b2]
                )

        return carry

    lax.fori_loop(0, _NCHUNK // _NBUF, outer, 0)

    for j in range(_NCHUNK - _NBUF, _NCHUNK):
        b = j % _NBUF
        pltpu.make_async_copy(rows[b], _out_slice(out_hbm, base, j), osem[b]).wait()


_sc_gather = functools.partial(
    pl.kernel,
    out_type=jax.ShapeDtypeStruct((_T, _H), jnp.float32),
    mesh=plsc.VectorSubcoreMesh(core_axis_name="c", subcore_axis_name="s"),
    scratch_types=[
        pltpu.VMEM((_NCHUNK, _C), jnp.int32),
        pltpu.VMEM_SHARED((3 * _S, _H), jnp.float32),
    ]
    + [pltpu.VMEM((_C, _H), jnp.float32) for _ in range(_NBUF)]
    + [pltpu.SemaphoreType.DMA for _ in range(2 * _NBUF)],
)(_sc_body)


def kernel(interaction, emb_interaction, emb_position):
    flat, comb3 = _prep(interaction, emb_interaction, emb_position)
    out_flat = _sc_gather(comb3.reshape(3 * _S, _H), flat.reshape(_T // _C, _C))
    return out_flat.reshape(_B, _S, _H)


# speedup vs baseline: 1.0561x; 1.0561x over previous
import functools

import jax
import jax.numpy as jnp
from jax import lax
from jax.experimental import pallas as pl
from jax.experimental.pallas import tpu as pltpu
from jax.experimental.pallas import tpu_sc as plsc

_B = 4096
_S = 200
_H = 128
_T = _B * _S
_BB = 128

_NC = 2
_NS = 16
_NW = _NC * _NS
_PW = _T // _NW
_C = 128
_NCHUNK = _PW // _C


def _prep_body(idx_ref, emb_int_ref, emb_pos_ref, flat_ref, comb_ref):
    i = pl.program_id(0)
    pos = emb_pos_ref[...]

    @pl.when(i == 0)
    def _():
        comb_ref[...] = pos[None, :, :] + emb_int_ref[...][:, None, :]

    flat_ref[...] = idx_ref[...] * _S + lax.broadcasted_iota(jnp.int32, (_BB, _S), 1)


def _prep(interaction, emb_interaction, emb_position):
    return pl.pallas_call(
        _prep_body,
        grid=(_B // _BB,),
        in_specs=[
            pl.BlockSpec((_BB, _S), lambda i: (i, 0)),
            pl.BlockSpec((3, _H), lambda i: (0, 0)),
            pl.BlockSpec((_S, _H), lambda i: (0, 0)),
        ],
        out_specs=[
            pl.BlockSpec((_BB, _S), lambda i: (i, 0)),
            pl.BlockSpec((3, _S, _H), lambda i: (0, 0, 0)),
        ],
        out_shape=[
            jax.ShapeDtypeStruct((_B, _S), jnp.int32),
            jax.ShapeDtypeStruct((3, _S, _H), jnp.float32),
        ],
    )(interaction, emb_interaction, emb_position)


_NBUF = 5
_LOOK = 2


def _out_slice(out_hbm, base, g):
    return out_hbm.at[pl.ds(base + g * _C, _C)]


def _sc_body(comb_hbm, flat_hbm, out_hbm, idx_all, comb_sp, *scratch):
    rows = scratch[:_NBUF]
    gsem = scratch[_NBUF : 2 * _NBUF]
    osem = scratch[2 * _NBUF : 3 * _NBUF]
    sid = lax.axis_index("s")
    wid = sid * _NC + lax.axis_index("c")
    base = wid * _PW

    @pl.when(sid == 0)
    def _():
        pltpu.sync_copy(comb_hbm, comb_sp)

    pltpu.sync_copy(flat_hbm.at[pl.ds(wid * _NCHUNK, _NCHUNK)], idx_all)
    plsc.subcore_barrier()

    for b in range(_LOOK):
        pltpu.async_copy(comb_sp.at[idx_all.at[b]], rows[b], gsem[b])

    def outer(i, carry):
        for b in range(_NBUF):
            g = i * _NBUF + b
            pltpu.make_async_copy(comb_sp.at[idx_all.at[g]], rows[b], gsem[b]).wait()
            pltpu.async_copy(rows[b], _out_slice(out_hbm, base, g), osem[b])
            b2 = (b + _LOOK) % _NBUF

            @pl.when(g + _LOOK < _NCHUNK)
            def _():
                @pl.when(g + _LOOK >= _NBUF)
                def _():
                    pltpu.make_async_copy(
                        rows[b2], _out_slice(out_hbm, base, g + _LOOK - _NBUF), osem[b2]
                    ).wait()

                pltpu.async_copy(
                    comb_sp.at[idx_all.at[g + _LOOK]], rows[b2], gsem[b2]
                )

        return carry

    lax.fori_loop(0, _NCHUNK // _NBUF, outer, 0)

    for j in range(_NCHUNK - _NBUF, _NCHUNK):
        b = j % _NBUF
        pltpu.make_async_copy(rows[b], _out_slice(out_hbm, base, j), osem[b]).wait()


_sc_gather = functools.partial(
    pl.kernel,
    out_type=jax.ShapeDtypeStruct((_T, _H), jnp.float32),
    mesh=plsc.VectorSubcoreMesh(core_axis_name="c", subcore_axis_name="s"),
    scratch_types=[
        pltpu.VMEM((_NCHUNK, _C), jnp.int32),
        pltpu.VMEM_SHARED((3 * _S, _H), jnp.float32),
    ]
    + [pltpu.VMEM((_C, _H), jnp.float32) for _ in range(_NBUF)]
    + [pltpu.SemaphoreType.DMA for _ in range(2 * _NBUF)],
)(_sc_body)


def kernel(interaction, emb_interaction, emb_position):
    flat, comb3 = _prep(interaction, emb_interaction, emb_position)
    out_flat = _sc_gather(comb3.reshape(3 * _S, _H), flat.reshape(_T // _C, _C))
    return out_flat.reshape(_B, _S, _H)


# speedup vs baseline: 1.1463x; 1.0854x over previous
import functools

import jax
import jax.numpy as jnp
from jax import lax
from jax.experimental import pallas as pl
from jax.experimental.pallas import tpu as pltpu
from jax.experimental.pallas import tpu_sc as plsc

_B = 4096
_S = 200
_H = 128
_T = _B * _S
_L = 16

_NC = 2
_NS = 16
_NW = _NC * _NS
_PW = _T // _NW
_C = 128
_NCHUNK = _PW // _C
_NBUF = 5
_LOOK = 2

_PROWS = 16


def _out_slice(out_hbm, base, g):
    return out_hbm.at[pl.ds(base + g * _C, _C)]


def _build_comb(sid, emb_int_hbm, emb_pos_hbm, e_int, pos_chunk, cmb_chunk, comb_sp):
    ss = jnp.minimum(sid * _PROWS, _S - _PROWS)
    pltpu.sync_copy(emb_int_hbm, e_int)
    pltpu.sync_copy(emb_pos_hbm.at[pl.ds(ss, _PROWS)], pos_chunk)
    for k in range(3):
        for i in range(_PROWS):
            for j in range(_H // _L):
                sl = pl.ds(j * _L, _L)
                cmb_chunk[i, sl] = pos_chunk[i, sl] + e_int[k, sl]
        pltpu.sync_copy(cmb_chunk, comb_sp.at[pl.ds(k * _S + ss, _PROWS)])


def _sc_body(ids_hbm, emb_int_hbm, emb_pos_hbm, out_hbm, *scratch):
    rows = scratch[:_NBUF]
    idx_all, e_int, pos_chunk, cmb_chunk, comb_sp = scratch[_NBUF : _NBUF + 5]
    sems = scratch[_NBUF + 5 :]
    gsem = sems[:_NBUF]
    osem = sems[_NBUF : 2 * _NBUF]
    isem = sems[2 * _NBUF]

    sid = lax.axis_index("s")
    wid = sid * _NC + lax.axis_index("c")
    base = wid * _PW

    ids_in = pltpu.make_async_copy(
        ids_hbm.at[pl.ds(wid * _NCHUNK, _NCHUNK)], idx_all, isem
    )
    ids_in.start()
    _build_comb(sid, emb_int_hbm, emb_pos_hbm, e_int, pos_chunk, cmb_chunk, comb_sp)
    ids_in.wait()

    def flatten(g, carry):
        for j in range(_C // _L):
            sl = pl.ds(j * _L, _L)
            t = lax.broadcasted_iota(jnp.int32, (_L,), 0) + (base + g * _C + j * _L)
            idx_all[g, sl] = idx_all[g, sl] * _S + lax.rem(t, _S)
        return carry

    lax.fori_loop(0, _NCHUNK, flatten, 0)
    plsc.subcore_barrier()

    def gather_dma(g, b):
        return pltpu.make_async_copy(comb_sp.at[idx_all.at[g]], rows[b], gsem[b])

    for g in range(_LOOK):
        gather_dma(g, g).start()

    def outer(i, carry):
        for b in range(_NBUF):
            g = i * _NBUF + b
            gather_dma(g, b).wait()
            pltpu.async_copy(rows[b], _out_slice(out_hbm, base, g), osem[b])

            b2 = (b + _LOOK) % _NBUF

            @pl.when(g + _LOOK < _NCHUNK)
            def _():
                @pl.when(g + _LOOK >= _NBUF)
                def _():
                    pltpu.make_async_copy(
                        rows[b2], _out_slice(out_hbm, base, g + _LOOK - _NBUF), osem[b2]
                    ).wait()

                gather_dma(g + _LOOK, b2).start()

        return carry

    lax.fori_loop(0, _NCHUNK // _NBUF, outer, 0)

    for j in range(_NCHUNK - _NBUF, _NCHUNK):
        b = j % _NBUF
        pltpu.make_async_copy(rows[b], _out_slice(out_hbm, base, j), osem[b]).wait()


_sc_kernel = functools.partial(
    pl.kernel,
    out_type=jax.ShapeDtypeStruct((_T, _H), jnp.float32),
    mesh=plsc.VectorSubcoreMesh(core_axis_name="c", subcore_axis_name="s"),
    scratch_types=[pltpu.VMEM((_C, _H), jnp.float32) for _ in range(_NBUF)]
    + [
        pltpu.VMEM((_NCHUNK, _C), jnp.int32),
        pltpu.VMEM((3, _H), jnp.float32),
        pltpu.VMEM((_PROWS, _H), jnp.float32),
        pltpu.VMEM((_PROWS, _H), jnp.float32),
        pltpu.VMEM_SHARED((3 * _S, _H), jnp.float32),
    ]
    + [pltpu.SemaphoreType.DMA for _ in range(2 * _NBUF + 1)],
)(_sc_body)


def kernel(interaction, emb_interaction, emb_position):
    out_flat = _sc_kernel(
        interaction.reshape(_T // _C, _C), emb_interaction, emb_position
    )
    return out_flat.reshape(_B, _S, _H)


# speedup vs baseline: 1.1506x; 1.0038x over previous
import functools

import jax
import jax.numpy as jnp
from jax import lax
from jax.experimental import pallas as pl
from jax.experimental.pallas import tpu as pltpu
from jax.experimental.pallas import tpu_sc as plsc

_B = 4096
_S = 200
_H = 128
_T = _B * _S
_L = 16

_NC = 2
_NS = 16
_NW = _NC * _NS
_PW = _T // _NW
_C = 128
_NCHUNK = _PW // _C
_NBUF = 6
_LOOK = 2
_LOOPN = (_NCHUNK // _NBUF) * _NBUF

_PROWS = 16


def _out_slice(out_hbm, base, g):
    return out_hbm.at[pl.ds(base + g * _C, _C)]


def _build_comb(sid, emb_int_hbm, emb_pos_hbm, e_int, pos_chunk, cmb_chunk, comb_sp):
    ss = jnp.minimum(sid * _PROWS, _S - _PROWS)
    pltpu.sync_copy(emb_int_hbm, e_int.at[pl.ds(0, 3)])
    pltpu.sync_copy(emb_pos_hbm.at[pl.ds(ss, _PROWS)], pos_chunk.at[pl.ds(0, _PROWS)])
    for k in range(3):
        for i in range(_PROWS):
            for j in range(_H // _L):
                sl = pl.ds(j * _L, _L)
                cmb_chunk[i, sl] = pos_chunk[i, sl] + e_int[k, sl]
        pltpu.sync_copy(
            cmb_chunk.at[pl.ds(0, _PROWS)], comb_sp.at[pl.ds(k * _S + ss, _PROWS)]
        )


def _sc_body(ids_hbm, emb_int_hbm, emb_pos_hbm, out_hbm, *scratch):
    rows = scratch[:_NBUF]
    idx_all, comb_sp = scratch[_NBUF : _NBUF + 2]
    sems = scratch[_NBUF + 2 :]
    gsem = sems[:_NBUF]
    osem = sems[_NBUF : 2 * _NBUF]
    isem = sems[2 * _NBUF]

    sid = lax.axis_index("s")
    wid = sid * _NC + lax.axis_index("c")
    base = wid * _PW

    ids_in = pltpu.make_async_copy(
        ids_hbm.at[pl.ds(wid * _NCHUNK, _NCHUNK)], idx_all, isem
    )
    ids_in.start()
    _build_comb(sid, emb_int_hbm, emb_pos_hbm, rows[1], rows[2], rows[0], comb_sp)
    ids_in.wait()

    def flatten(g, carry):
        for j in range(_C // _L):
            sl = pl.ds(j * _L, _L)
            t = lax.broadcasted_iota(jnp.int32, (_L,), 0) + (base + g * _C + j * _L)
            idx_all[g, sl] = idx_all[g, sl] * _S + lax.rem(t, _S)
        return carry

    lax.fori_loop(0, _NCHUNK, flatten, 0)
    plsc.subcore_barrier()

    def gather_dma(g, b):
        return pltpu.make_async_copy(comb_sp.at[idx_all.at[g]], rows[b], gsem[b])

    for g in range(_LOOK):
        gather_dma(g, g).start()

    def outer(i, carry):
        for b in range(_NBUF):
            g = i * _NBUF + b
            gather_dma(g, b).wait()
            pltpu.async_copy(rows[b], _out_slice(out_hbm, base, g), osem[b])

            b2 = (b + _LOOK) % _NBUF

            @pl.when(g + _LOOK < _NCHUNK)
            def _():
                @pl.when(g + _LOOK >= _NBUF)
                def _():
                    pltpu.make_async_copy(
                        rows[b2], _out_slice(out_hbm, base, g + _LOOK - _NBUF), osem[b2]
                    ).wait()

                gather_dma(g + _LOOK, b2).start()

        return carry

    lax.fori_loop(0, _LOOPN // _NBUF, outer, 0)

    for g in range(_LOOPN, _NCHUNK):
        b = g % _NBUF
        gather_dma(g, b).wait()
        pltpu.async_copy(rows[b], _out_slice(out_hbm, base, g), osem[b])

    for j in range(_NCHUNK - _NBUF, _NCHUNK):
        b = j % _NBUF
        pltpu.make_async_copy(rows[b], _out_slice(out_hbm, base, j), osem[b]).wait()


_sc_kernel = functools.partial(
    pl.kernel,
    out_type=jax.ShapeDtypeStruct((_T, _H), jnp.float32),
    mesh=plsc.VectorSubcoreMesh(core_axis_name="c", subcore_axis_name="s"),
    scratch_types=[pltpu.VMEM((_C, _H), jnp.float32) for _ in range(_NBUF)]
    + [
        pltpu.VMEM((_NCHUNK, _C), jnp.int32),
        pltpu.VMEM_SHARED((3 * _S, _H), jnp.float32),
    ]
    + [pltpu.SemaphoreType.DMA for _ in range(2 * _NBUF + 1)],
)(_sc_body)


def kernel(interaction, emb_interaction, emb_position):
    out_flat = _sc_kernel(
        interaction.reshape(_T // _C, _C), emb_interaction, emb_position
    )
    return out_flat.reshape(_B, _S, _H)
